# Initial kernel scaffold; baseline (speedup 1.0000x reference)
import jax
import jax.numpy as jnp
from jax.experimental import pallas as pl


def kernel(x, a_hat, w11, b11, w12, b12, w13, b13, w21, b21, w22, b22, w23, b23, theta, gamma, beta):
    raise NotImplementedError("write your pallas kernel here")


# speedup vs baseline: 2.2435x; 2.2435x over previous
import jax
import jax.numpy as jnp
from jax.experimental import pallas as pl
from jax.experimental.pallas import tpu as pltpu


def _stgcn_forward(x, a_hat, w1m, b1, w2m, b2, theta, gamma, beta):
    B, N, T, Cin = x.shape
    K1 = 3
    T1 = T - K1 + 1
    Cout = w1m.shape[1] // 3
    Csp = theta.shape[1]
    K2 = 3
    T2 = T1 - K2 + 1
    Cout2 = w2m.shape[1] // 3

    xb = x.astype(jnp.bfloat16)
    w1b = w1m.astype(jnp.bfloat16)
    w2b = w2m.astype(jnp.bfloat16)
    thb = theta.astype(jnp.bfloat16)
    ab = a_hat.astype(jnp.bfloat16)

    def k1(x_ref, w_ref, b_ref, th_ref, a_ref, o_ref):
        xk = x_ref[0]
        xcat = jnp.concatenate([xk[:, k:k + T1, :] for k in range(K1)],
                               axis=-1)
        xm = xcat.reshape(N * T1, K1 * Cin)
        y = jnp.dot(xm, w_ref[...],
                    preferred_element_type=jnp.float32) + b_ref[...]
        c1 = y[:, :Cout]
        c2 = y[:, Cout:2 * Cout]
        c3 = y[:, 2 * Cout:]
        t = jnp.maximum(c1 + jax.nn.sigmoid(c2) + c3, 0.0)
        u = jnp.dot(t.astype(jnp.bfloat16), th_ref[...],
                    preferred_element_type=jnp.float32)
        u2 = u.astype(jnp.bfloat16).reshape(N, T1 * Csp)
        lfs = jnp.dot(a_ref[...], u2,
                      preferred_element_type=jnp.float32)
        o_ref[0] = jnp.maximum(lfs, 0.0).astype(jnp.bfloat16)

    t2 = pl.pallas_call(
        k1,
        grid=(B,),
        in_specs=[
            pl.BlockSpec((1, N, T, Cin), lambda b: (b, 0, 0, 0)),
            pl.BlockSpec((K1 * Cin, 3 * Cout), lambda b: (0, 0)),
            pl.BlockSpec((1, 3 * Cout), lambda b: (0, 0)),
            pl.BlockSpec((Cout, Csp), lambda b: (0, 0)),
            pl.BlockSpec((N, N), lambda b: (0, 0)),
        ],
        out_specs=pl.BlockSpec((1, N, T1 * Csp), lambda b: (b, 0, 0)),
        out_shape=jax.ShapeDtypeStruct((B, N, T1 * Csp), jnp.bfloat16),
        compiler_params=pltpu.CompilerParams(dimension_semantics=("parallel",)),
    )(xb, w1b, b1, thb, ab)

    NB = 8
    while N % NB:
        NB //= 2

    def k2(t2_ref, w_ref, b_ref, g_ref, bt_ref, o_ref):
        n0 = pl.program_id(0)
        z = t2_ref[...].reshape(B * NB, T1, Csp)
        zcat = jnp.concatenate([z[:, k:k + T2, :] for k in range(K2)],
                               axis=-1)
        zm = zcat.reshape(B * NB * T2, K2 * Csp)
        y = jnp.dot(zm, w_ref[...],
                    preferred_element_type=jnp.float32) + b_ref[...]
        c1 = y[:, :Cout2]
        c2 = y[:, Cout2:2 * Cout2]
        c3 = y[:, 2 * Cout2:]
        t3 = jnp.maximum(c1 + jax.nn.sigmoid(c2) + c3, 0.0)
        t4 = t3.reshape(B, NB, T2, Cout2)
        mean = jnp.mean(t4, axis=(0, 2, 3), keepdims=True)
        cen = t4 - mean
        var = jnp.mean(cen * cen, axis=(0, 2, 3), keepdims=True)
        inv = jax.lax.rsqrt(var + 1e-5)
        for i in range(NB):
            g = g_ref[n0 * NB + i]
            be = bt_ref[n0 * NB + i]
            o_ref[:, i] = cen[:, i] * (inv[0, i] * g) + be

    out = pl.pallas_call(
        k2,
        grid=(N // NB,),
        in_specs=[
            pl.BlockSpec((B, NB, T1 * Csp), lambda n: (0, n, 0)),
            pl.BlockSpec((K2 * Csp, 3 * Cout2), lambda n: (0, 0)),
            pl.BlockSpec((1, 3 * Cout2), lambda n: (0, 0)),
            pl.BlockSpec(memory_space=pltpu.MemorySpace.SMEM),
            pl.BlockSpec(memory_space=pltpu.MemorySpace.SMEM),
        ],
        out_specs=pl.BlockSpec((B, NB, T2, Cout2), lambda n: (0, n, 0, 0)),
        out_shape=jax.ShapeDtypeStruct((B, N, T2, Cout2), jnp.float32),
        compiler_params=pltpu.CompilerParams(dimension_semantics=("parallel",)),
    )(t2, w2b, b2, gamma, beta)
    return out


def kernel(x, a_hat, w11, b11, w12, b12, w13, b13,
           w21, b21, w22, b22, w23, b23, theta, gamma, beta):
    K1, Cin, Cout = w11.shape
    K2, Csp, Cout2 = w21.shape
    w1m = jnp.concatenate([w11, w12, w13], axis=-1).reshape(K1 * Cin, 3 * Cout)
    b1 = jnp.concatenate([b11, b12, b13], axis=-1)
    w2m = jnp.concatenate([w21, w22, w23], axis=-1).reshape(K2 * Csp, 3 * Cout2)
    b2 = jnp.concatenate([b21, b22, b23], axis=-1)
    return _stgcn_forward(x, a_hat, w1m, b1, w2m, b2, theta, gamma, beta)


# speedup vs baseline: 2.4120x; 1.0751x over previous
import jax
import jax.numpy as jnp
from jax.experimental import pallas as pl
from jax.experimental.pallas import tpu as pltpu


def _stgcn_forward(x, a_hat, w1m, b1, w2m, b2, theta, gamma, beta):
    B, N, T, Cin = x.shape
    K1 = 3
    T1 = T - K1 + 1
    Cout = w1m.shape[-1] // 3
    Csp = theta.shape[1]
    K2 = 3
    T2 = T1 - K2 + 1
    Cout2 = w2m.shape[1] // 3

    w1b = w1m.astype(jnp.bfloat16)
    w2b = w2m.astype(jnp.bfloat16)
    thb = theta.astype(jnp.bfloat16)
    ab = a_hat.astype(jnp.bfloat16)

    BB = 4
    while B % BB:
        BB //= 2

    def k1(x_ref, w_ref, b_ref, th_ref, a_ref, o_ref):
        xk = x_ref[...].astype(jnp.bfloat16).reshape(BB * N, T, Cin)
        xcat = jnp.concatenate([xk[:, k:k + T1, :] for k in range(K1)],
                               axis=-1)
        xm = xcat.reshape(BB * N * T1, K1 * Cin)
        y = jnp.dot(xm, w_ref[...],
                    preferred_element_type=jnp.float32) + b_ref[...]
        c1 = y[:, :Cout]
        c2 = y[:, Cout:2 * Cout]
        c3 = y[:, 2 * Cout:]
        t = jnp.maximum(c1 + jax.nn.sigmoid(c2) + c3, 0.0)
        u = jnp.dot(t.astype(jnp.bfloat16), th_ref[...],
                    preferred_element_type=jnp.float32)
        u2 = u.astype(jnp.bfloat16).reshape(BB, N, T1 * Csp)
        for i in range(BB):
            lfs = jnp.dot(a_ref[...], u2[i],
                          preferred_element_type=jnp.float32)
            o_ref[i] = jnp.maximum(lfs, 0.0).astype(jnp.bfloat16)

    t2 = pl.pallas_call(
        k1,
        grid=(B // BB,),
        in_specs=[
            pl.BlockSpec((BB, N, T, Cin), lambda b: (b, 0, 0, 0)),
            pl.BlockSpec((K1 * Cin, 3 * Cout), lambda b: (0, 0)),
            pl.BlockSpec((1, 3 * Cout), lambda b: (0, 0)),
            pl.BlockSpec((Cout, Csp), lambda b: (0, 0)),
            pl.BlockSpec((N, N), lambda b: (0, 0)),
        ],
        out_specs=pl.BlockSpec((BB, N, T1 * Csp), lambda b: (b, 0, 0)),
        out_shape=jax.ShapeDtypeStruct((B, N, T1 * Csp), jnp.bfloat16),
        compiler_params=pltpu.CompilerParams(dimension_semantics=("parallel",)),
    )(x, w1b, b1, thb, ab)

    NB = 16
    while N % NB:
        NB //= 2

    def k2(t2_ref, w_ref, b_ref, g_ref, bt_ref, o_ref):
        n0 = pl.program_id(0)
        z = t2_ref[...].reshape(B * NB, T1, Csp)
        zcat = jnp.concatenate([z[:, k:k + T2, :] for k in range(K2)],
                               axis=-1)
        zm = zcat.reshape(B * NB * T2, K2 * Csp)
        y = jnp.dot(zm, w_ref[...],
                    preferred_element_type=jnp.float32) + b_ref[...]
        c1 = y[:, :Cout2]
        c2 = y[:, Cout2:2 * Cout2]
        c3 = y[:, 2 * Cout2:]
        t3 = jnp.maximum(c1 + jax.nn.sigmoid(c2) + c3, 0.0)
        t4 = t3.reshape(B, NB, T2, Cout2)
        mean = jnp.mean(t4, axis=(0, 2, 3), keepdims=True)
        cen = t4 - mean
        var = jnp.mean(cen * cen, axis=(0, 2, 3), keepdims=True)
        inv = jax.lax.rsqrt(var + 1e-5)
        for i in range(NB):
            g = g_ref[n0 * NB + i]
            be = bt_ref[n0 * NB + i]
            o_ref[:, i] = cen[:, i] * (inv[0, i] * g) + be

    out = pl.pallas_call(
        k2,
        grid=(N // NB,),
        in_specs=[
            pl.BlockSpec((B, NB, T1 * Csp), lambda n: (0, n, 0)),
            pl.BlockSpec((K2 * Csp, 3 * Cout2), lambda n: (0, 0)),
            pl.BlockSpec((1, 3 * Cout2), lambda n: (0, 0)),
            pl.BlockSpec(memory_space=pltpu.MemorySpace.SMEM),
            pl.BlockSpec(memory_space=pltpu.MemorySpace.SMEM),
        ],
        out_specs=pl.BlockSpec((B, NB, T2, Cout2), lambda n: (0, n, 0, 0)),
        out_shape=jax.ShapeDtypeStruct((B, N, T2, Cout2), jnp.float32),
        compiler_params=pltpu.CompilerParams(dimension_semantics=("parallel",)),
    )(t2, w2b, b2, gamma, beta)
    return out


def kernel(x, a_hat, w11, b11, w12, b12, w13, b13,
           w21, b21, w22, b22, w23, b23, theta, gamma, beta):
    K1, Cin, Cout = w11.shape
    K2, Csp, Cout2 = w21.shape
    w1m = jnp.concatenate([w11, w12, w13], axis=-1).reshape(K1 * Cin, 3 * Cout)
    b1 = jnp.concatenate([b11, b12, b13], axis=-1)
    w2m = jnp.concatenate([w21, w22, w23], axis=-1).reshape(K2 * Csp, 3 * Cout2)
    b2 = jnp.concatenate([b21, b22, b23], axis=-1)
    return _stgcn_forward(x, a_hat, w1m, b1, w2m, b2, theta, gamma, beta)


# speedup vs baseline: 2.5931x; 1.0751x over previous
import jax
import jax.numpy as jnp
from jax.experimental import pallas as pl
from jax.experimental.pallas import tpu as pltpu


def _stgcn_forward(x, a_hat, w1m, b1, w2m, b2, theta, gamma, beta):
    B, N, T, Cin = x.shape
    K1 = 3
    T1 = T - K1 + 1
    Cout = w1m.shape[-1] // 2
    Csp = theta.shape[1]
    K2 = 3
    T2 = T1 - K2 + 1
    Cout2 = w2m.shape[1] // 2

    w1b = w1m.astype(jnp.bfloat16)
    w2b = w2m.astype(jnp.bfloat16)
    thb = theta.astype(jnp.bfloat16)
    ab = a_hat.astype(jnp.bfloat16)

    BB = 4
    while B % BB:
        BB //= 2

    def k1(x_ref, w_ref, b_ref, th_ref, a_ref, o_ref):
        xk = x_ref[...].astype(jnp.bfloat16).reshape(BB * N, T, Cin)
        xcat = jnp.concatenate([xk[:, k:k + T1, :] for k in range(K1)],
                               axis=-1)
        xm = xcat.reshape(BB * N * T1, K1 * Cin)
        y = jnp.dot(xm, w_ref[...],
                    preferred_element_type=jnp.float32) + b_ref[...]
        t = jnp.maximum(y[:, :Cout] + jax.nn.sigmoid(y[:, Cout:]), 0.0)
        u = jnp.dot(t.astype(jnp.bfloat16), th_ref[...],
                    preferred_element_type=jnp.float32)
        u2 = u.astype(jnp.bfloat16).reshape(BB, N, T1 * Csp)
        for i in range(BB):
            lfs = jnp.dot(a_ref[...], u2[i],
                          preferred_element_type=jnp.float32)
            o_ref[i] = jnp.maximum(lfs, 0.0).astype(jnp.bfloat16)

    t2 = pl.pallas_call(
        k1,
        grid=(B // BB,),
        in_specs=[
            pl.BlockSpec((BB, N, T, Cin), lambda b: (b, 0, 0, 0)),
            pl.BlockSpec((K1 * Cin, 2 * Cout), lambda b: (0, 0)),
            pl.BlockSpec((1, 2 * Cout), lambda b: (0, 0)),
            pl.BlockSpec((Cout, Csp), lambda b: (0, 0)),
            pl.BlockSpec((N, N), lambda b: (0, 0)),
        ],
        out_specs=pl.BlockSpec((BB, N, T1 * Csp), lambda b: (b, 0, 0)),
        out_shape=jax.ShapeDtypeStruct((B, N, T1 * Csp), jnp.bfloat16),
        compiler_params=pltpu.CompilerParams(dimension_semantics=("parallel",)),
    )(x, w1b, b1, thb, ab)

    NB = 16
    while N % NB:
        NB //= 2

    def k2(t2_ref, w_ref, b_ref, g_ref, bt_ref, o_ref):
        n0 = pl.program_id(0)
        z = t2_ref[...].reshape(B * NB, T1, Csp)
        zcat = jnp.concatenate([z[:, k:k + T2, :] for k in range(K2)],
                               axis=-1)
        zm = zcat.reshape(B * NB * T2, K2 * Csp)
        y = jnp.dot(zm, w_ref[...],
                    preferred_element_type=jnp.float32) + b_ref[...]
        t3 = jnp.maximum(y[:, :Cout2] + jax.nn.sigmoid(y[:, Cout2:]), 0.0)
        t4 = t3.reshape(B, NB, T2, Cout2)
        mean = jnp.mean(t4, axis=(0, 2, 3), keepdims=True)
        cen = t4 - mean
        var = jnp.mean(cen * cen, axis=(0, 2, 3), keepdims=True)
        inv = jax.lax.rsqrt(var + 1e-5)
        for i in range(NB):
            g = g_ref[n0 * NB + i]
            be = bt_ref[n0 * NB + i]
            o_ref[:, i] = cen[:, i] * (inv[0, i] * g) + be

    out = pl.pallas_call(
        k2,
        grid=(N // NB,),
        in_specs=[
            pl.BlockSpec((B, NB, T1 * Csp), lambda n: (0, n, 0)),
            pl.BlockSpec((K2 * Csp, 2 * Cout2), lambda n: (0, 0)),
            pl.BlockSpec((1, 2 * Cout2), lambda n: (0, 0)),
            pl.BlockSpec(memory_space=pltpu.MemorySpace.SMEM),
            pl.BlockSpec(memory_space=pltpu.MemorySpace.SMEM),
        ],
        out_specs=pl.BlockSpec((B, NB, T2, Cout2), lambda n: (0, n, 0, 0)),
        out_shape=jax.ShapeDtypeStruct((B, N, T2, Cout2), jnp.float32),
        compiler_params=pltpu.CompilerParams(dimension_semantics=("parallel",)),
    )(t2, w2b, b2, gamma, beta)
    return out


def kernel(x, a_hat, w11, b11, w12, b12, w13, b13,
           w21, b21, w22, b22, w23, b23, theta, gamma, beta):
    K1, Cin, Cout = w11.shape
    K2, Csp, Cout2 = w21.shape
    w1m = jnp.concatenate([w11 + w13, w12], axis=-1).reshape(K1 * Cin, 2 * Cout)
    b1 = jnp.concatenate([b11 + b13, b12], axis=-1)
    w2m = jnp.concatenate([w21 + w23, w22], axis=-1).reshape(K2 * Csp, 2 * Cout2)
    b2 = jnp.concatenate([b21 + b23, b22], axis=-1)
    return _stgcn_forward(x, a_hat, w1m, b1, w2m, b2, theta, gamma, beta)
